# Initial kernel scaffold; baseline (speedup 1.0000x reference)
import jax
import jax.numpy as jnp
from jax.experimental import pallas as pl


def kernel(x, Wg, W1, b1, W2, b2, W3, b3, Ws1, bs1, Ws2, bs2, Ws3, bs3):
    raise NotImplementedError("write your pallas kernel here")


# speedup vs baseline: 1.6322x; 1.6322x over previous
import functools

import jax
import jax.numpy as jnp
from jax import lax
from jax.experimental import pallas as pl
from jax.experimental.pallas import tpu as pltpu
from jax.experimental.pallas import tpu_sc as plsc

T = 2048
DIM = 2048
E = 8
INTER = 1408
NSH = 2
BLK = 256
NB = (T * 2) // BLK + E - 1
NBL = 32
NRPAD = NB * BLK
_F32 = jnp.float32


def _sigmoid(v):
    return 1.0 / (1.0 + jnp.exp(-v))


def _gate_body(x_ref, wg_ref, wa_ref, wb_ref, d0_ref, d1_ref, emap_ref, act_ref):
    xv = x_ref[...]
    logits = lax.dot_general(xv, wg_ref[...], (((1,), (1,)), ((), ())),
                             preferred_element_type=_F32)
    m = jnp.max(logits, axis=1, keepdims=True)
    p = jnp.exp(logits - m)
    scores = p / jnp.sum(p, axis=1, keepdims=True)
    lane = lax.broadcasted_iota(jnp.int32, (T, E), 1)
    m1 = jnp.max(scores, axis=1, keepdims=True)
    e0 = jnp.min(jnp.where(scores == m1, lane, E), axis=1, keepdims=True)
    s2 = jnp.where(lane == e0, -1.0, scores)
    m2 = jnp.max(s2, axis=1, keepdims=True)
    e1 = jnp.min(jnp.where(s2 == m2, lane, E), axis=1, keepdims=True)
    wa_ref[...] = m1
    wb_ref[...] = m2
    cnt = ((lane == e0) | (lane == e1)).astype(_F32)
    C = 512
    rr = lax.broadcasted_iota(jnp.int32, (C, C), 0)
    cc = lax.broadcasted_iota(jnp.int32, (C, C), 1)
    tri = (cc < rr).astype(_F32)
    base = jnp.zeros((1, E), _F32)
    excl_parts = []
    for j in range(T // C):
        Cj = lax.slice(cnt, (j * C, 0), ((j + 1) * C, E))
        excl_parts.append(
            lax.dot_general(tri, Cj, (((1,), (0,)), ((), ())),
                            preferred_element_type=_F32) + base)
        base = base + jnp.sum(Cj, axis=0, keepdims=True)
    excl = jnp.concatenate(excl_parts, axis=0)
    counts = base
    pci = ((counts.astype(jnp.int32) + (BLK - 1)) // BLK) * BLK
    pcf = pci.astype(_F32)
    r8 = lax.broadcasted_iota(jnp.int32, (E, E), 0)
    c8 = lax.broadcasted_iota(jnp.int32, (E, E), 1)
    tri8 = (r8 < c8).astype(_F32)
    po = lax.dot_general(pcf, tri8, (((1,), (0,)), ((), ())),
                         preferred_element_type=_F32)
    d0 = jnp.sum(jnp.where(lane == e0, po + excl, 0.0), axis=1, keepdims=True)
    d1 = jnp.sum(jnp.where(lane == e1, po + excl, 0.0), axis=1, keepdims=True)
    d0_ref[...] = d0.astype(jnp.int32)
    d1_ref[...] = d1.astype(jnp.int32)
    ends = (po + pcf).astype(jnp.int32) // BLK
    bb = lax.broadcasted_iota(jnp.int32, (1, NBL), 1)
    eb = jnp.zeros((1, NBL), jnp.int32)
    for e in range(E):
        eb = eb + (bb >= ends[:, e:e + 1]).astype(jnp.int32)
    emap_ref[...] = jnp.minimum(eb, E - 1)
    nb_tot = jnp.sum(pci, axis=1, keepdims=True) // BLK
    act_ref[...] = (bb < nb_tot).astype(jnp.int32)


def _gate(x, Wg):
    return pl.pallas_call(
        _gate_body,
        out_shape=[
            jax.ShapeDtypeStruct((T, 1), _F32),
            jax.ShapeDtypeStruct((T, 1), _F32),
            jax.ShapeDtypeStruct((T, 1), jnp.int32),
            jax.ShapeDtypeStruct((T, 1), jnp.int32),
            jax.ShapeDtypeStruct((1, NBL), jnp.int32),
            jax.ShapeDtypeStruct((1, NBL), jnp.int32),
        ],
    )(x, Wg)


def _sc_scatter(x, d0, d1):
    mesh = plsc.VectorSubcoreMesh(core_axis_name="c", subcore_axis_name="s")
    nw = 32
    tpw = T // nw
    CH = 32

    @functools.partial(
        pl.kernel,
        out_type=jax.ShapeDtypeStruct((NRPAD, DIM), _F32),
        mesh=mesh,
        scratch_types=[
            pltpu.VMEM((CH,), jnp.int32),
            pltpu.VMEM((CH,), jnp.int32),
            pltpu.VMEM((CH, DIM), _F32),
            pltpu.SemaphoreType.DMA,
        ],
    )
    def k(x_hbm, d0_hbm, d1_hbm, xs_hbm, i0_v, i1_v, rows_v, sem):
        wid = lax.axis_index("s") * 2 + lax.axis_index("c")
        for c2 in range(tpw // CH):
            base = wid * tpw + c2 * CH
            pltpu.sync_copy(d0_hbm.at[pl.ds(base, CH)], i0_v)
            pltpu.sync_copy(d1_hbm.at[pl.ds(base, CH)], i1_v)
            pltpu.sync_copy(x_hbm.at[pl.ds(base, CH)], rows_v)
            cp0 = pltpu.async_copy(rows_v, xs_hbm.at[i0_v], sem)
            cp1 = pltpu.async_copy(rows_v, xs_hbm.at[i1_v], sem)
            cp0.wait()
            cp1.wait()

    return k(x, d0, d1)


def _sc_gather(r, d0, d1):
    mesh = plsc.VectorSubcoreMesh(core_axis_name="c", subcore_axis_name="s")
    nw = 32
    tpw = T // nw
    CH = 32

    @functools.partial(
        pl.kernel,
        out_type=[
            jax.ShapeDtypeStruct((T, DIM), _F32),
            jax.ShapeDtypeStruct((T, DIM), _F32),
        ],
        mesh=mesh,
        scratch_types=[
            pltpu.VMEM((CH,), jnp.int32),
            pltpu.VMEM((CH,), jnp.int32),
            pltpu.VMEM((CH, DIM), _F32),
            pltpu.SemaphoreType.DMA,
        ],
    )
    def k(r_hbm, d0_hbm, d1_hbm, o0_hbm, o1_hbm, i0_v, i1_v, rows_v, sem):
        wid = lax.axis_index("s") * 2 + lax.axis_index("c")
        for c2 in range(tpw // CH):
            base = wid * tpw + c2 * CH
            pltpu.sync_copy(d0_hbm.at[pl.ds(base, CH)], i0_v)
            pltpu.async_copy(r_hbm.at[i0_v], rows_v, sem).wait()
            pltpu.sync_copy(rows_v, o0_hbm.at[pl.ds(base, CH)])
            pltpu.sync_copy(d1_hbm.at[pl.ds(base, CH)], i1_v)
            pltpu.async_copy(r_hbm.at[i1_v], rows_v, sem).wait()
            pltpu.sync_copy(rows_v, o1_hbm.at[pl.ds(base, CH)])

    return k(r, d0, d1)


def _mlp_in_body(emap_ref, act_ref, xs_ref, w1_ref, b1_ref, w3_ref, b3_ref, g_ref):
    @pl.when(act_ref[pl.program_id(0)] == 1)
    def _():
        xv = xs_ref[...]
        h1 = lax.dot_general(xv, w1_ref[0], (((1,), (1,)), ((), ())),
                             preferred_element_type=_F32) + b1_ref[0]
        h3 = lax.dot_general(xv, w3_ref[0], (((1,), (1,)), ((), ())),
                             preferred_element_type=_F32) + b3_ref[0]
        g_ref[...] = h1 * _sigmoid(h1) * h3


def _mlp_in(emap, act, xs, W1, b1, W3, b3):
    return pl.pallas_call(
        _mlp_in_body,
        grid_spec=pltpu.PrefetchScalarGridSpec(
            num_scalar_prefetch=2,
            grid=(NB,),
            in_specs=[
                pl.BlockSpec((BLK, DIM), lambda b, em, ac: (b, 0)),
                pl.BlockSpec((1, INTER, DIM), lambda b, em, ac: (em[b], 0, 0)),
                pl.BlockSpec((1, 1, INTER), lambda b, em, ac: (em[b], 0, 0)),
                pl.BlockSpec((1, INTER, DIM), lambda b, em, ac: (em[b], 0, 0)),
                pl.BlockSpec((1, 1, INTER), lambda b, em, ac: (em[b], 0, 0)),
            ],
            out_specs=pl.BlockSpec((BLK, INTER), lambda b, em, ac: (b, 0)),
        ),
        out_shape=jax.ShapeDtypeStruct((NRPAD, INTER), _F32),
    )(emap, act, xs, W1, b1, W3, b3)


def _mlp_out_body(emap_ref, act_ref, g_ref, w2_ref, b2_ref, r_ref):
    @pl.when(act_ref[pl.program_id(0)] == 1)
    def _():
        r_ref[...] = lax.dot_general(g_ref[...], w2_ref[0], (((1,), (1,)), ((), ())),
                                     preferred_element_type=_F32) + b2_ref[0]


def _mlp_out(emap, act, g, W2, b2):
    return pl.pallas_call(
        _mlp_out_body,
        grid_spec=pltpu.PrefetchScalarGridSpec(
            num_scalar_prefetch=2,
            grid=(NB,),
            in_specs=[
                pl.BlockSpec((BLK, INTER), lambda b, em, ac: (b, 0)),
                pl.BlockSpec((1, DIM, INTER), lambda b, em, ac: (em[b], 0, 0)),
                pl.BlockSpec((1, 1, DIM), lambda b, em, ac: (em[b], 0, 0)),
            ],
            out_specs=pl.BlockSpec((BLK, DIM), lambda b, em, ac: (b, 0)),
        ),
        out_shape=jax.ShapeDtypeStruct((NRPAD, DIM), _F32),
    )(emap, act, g, W2, b2)


def _sh_in_body(x_ref, ws1_ref, bs1_ref, ws3_ref, bs3_ref, g_ref):
    xv = x_ref[...]
    h1 = lax.dot_general(xv, ws1_ref[...], (((1,), (1,)), ((), ())),
                         preferred_element_type=_F32) + bs1_ref[0]
    h3 = lax.dot_general(xv, ws3_ref[...], (((1,), (1,)), ((), ())),
                         preferred_element_type=_F32) + bs3_ref[0]
    g_ref[...] = h1 * _sigmoid(h1) * h3


def _sh_in(x, Ws1, bs1r, Ws3, bs3r):
    return pl.pallas_call(
        _sh_in_body,
        grid=(NSH, T // BLK),
        in_specs=[
            pl.BlockSpec((BLK, DIM), lambda i, b: (b, 0)),
            pl.BlockSpec((INTER, DIM), lambda i, b: (i, 0)),
            pl.BlockSpec((1, 1, INTER), lambda i, b: (i, 0, 0)),
            pl.BlockSpec((INTER, DIM), lambda i, b: (i, 0)),
            pl.BlockSpec((1, 1, INTER), lambda i, b: (i, 0, 0)),
        ],
        out_specs=pl.BlockSpec((BLK, INTER), lambda i, b: (b, i)),
        out_shape=jax.ShapeDtypeStruct((T, NSH * INTER), _F32),
    )(x, Ws1, bs1r, Ws3, bs3r)


def _sh_out_body(g_ref, ws2_ref, z_ref):
    z_ref[0] = lax.dot_general(g_ref[...], ws2_ref[...], (((1,), (1,)), ((), ())),
                               preferred_element_type=_F32)


def _sh_out(gs, Ws2):
    return pl.pallas_call(
        _sh_out_body,
        grid=(NSH, T // BLK),
        in_specs=[
            pl.BlockSpec((BLK, INTER), lambda i, b: (b, i)),
            pl.BlockSpec((DIM, INTER), lambda i, b: (0, i)),
        ],
        out_specs=pl.BlockSpec((1, BLK, DIM), lambda i, b: (i, b, 0)),
        out_shape=jax.ShapeDtypeStruct((NSH, T, DIM), _F32),
    )(gs, Ws2)


def _combine_body(r0_ref, r1_ref, z_ref, wa_ref, wb_ref, bs2_ref, y_ref):
    y_ref[...] = (wa_ref[...] * r0_ref[...] + wb_ref[...] * r1_ref[...]
                  + z_ref[0] + z_ref[1] + bs2_ref[...])


def _combine(r0, r1, z2, wa, wb, bs2r):
    return pl.pallas_call(
        _combine_body,
        grid=(T // BLK,),
        in_specs=[
            pl.BlockSpec((BLK, DIM), lambda b: (b, 0)),
            pl.BlockSpec((BLK, DIM), lambda b: (b, 0)),
            pl.BlockSpec((NSH, BLK, DIM), lambda b: (0, b, 0)),
            pl.BlockSpec((BLK, 1), lambda b: (b, 0)),
            pl.BlockSpec((BLK, 1), lambda b: (b, 0)),
            pl.BlockSpec((1, DIM), lambda b: (0, 0)),
        ],
        out_specs=pl.BlockSpec((BLK, DIM), lambda b: (b, 0)),
        out_shape=jax.ShapeDtypeStruct((T, DIM), _F32),
    )(r0, r1, z2, wa, wb, bs2r)


def kernel(x, Wg, W1, b1, W2, b2, W3, b3, Ws1, bs1, Ws2, bs2, Ws3, bs3):
    shape = x.shape
    xf = x.reshape(T, DIM)
    wa, wb, d0c, d1c, emap2, act2 = _gate(xf, Wg)
    emap = emap2.reshape(NBL)
    act = act2.reshape(NBL)
    d0 = d0c.reshape(T)
    d1 = d1c.reshape(T)
    xs = _sc_scatter(xf, d0, d1)
    g = _mlp_in(emap, act, xs, W1, b1.reshape(E, 1, INTER), W3, b3.reshape(E, 1, INTER))
    r = _mlp_out(emap, act, g, W2, b2.reshape(E, 1, DIM))
    r0, r1 = _sc_gather(r, d0, d1)
    gs = _sh_in(xf, Ws1, bs1.reshape(NSH, 1, INTER), Ws3, bs3.reshape(NSH, 1, INTER))
    z2 = _sh_out(gs, Ws2)
    y = _combine(r0, r1, z2, wa, wb, bs2.reshape(1, DIM))
    return y.reshape(shape)


# speedup vs baseline: 1.6333x; 1.0006x over previous
import functools

import jax
import jax.numpy as jnp
from jax import lax
from jax.experimental import pallas as pl
from jax.experimental.pallas import tpu as pltpu
from jax.experimental.pallas import tpu_sc as plsc

T = 2048
DIM = 2048
E = 8
INTER = 1408
NSH = 2
BLK = 256
NB = (T * 2) // BLK + E - 1
NBL = 32
NRPAD = NB * BLK
_F32 = jnp.float32
_BF = jnp.bfloat16


def _sigmoid(v):
    return 1.0 / (1.0 + jnp.exp(-v))


def _gate_body(x_ref, wg_ref, wa_ref, wb_ref, d0_ref, d1_ref, emap_ref, act_ref):
    xv = x_ref[...]
    logits = lax.dot_general(xv, wg_ref[...], (((1,), (1,)), ((), ())),
                             preferred_element_type=_F32)
    m = jnp.max(logits, axis=1, keepdims=True)
    p = jnp.exp(logits - m)
    scores = p / jnp.sum(p, axis=1, keepdims=True)
    lane = lax.broadcasted_iota(jnp.int32, (T, E), 1)
    m1 = jnp.max(scores, axis=1, keepdims=True)
    e0 = jnp.min(jnp.where(scores == m1, lane, E), axis=1, keepdims=True)
    s2 = jnp.where(lane == e0, -1.0, scores)
    m2 = jnp.max(s2, axis=1, keepdims=True)
    e1 = jnp.min(jnp.where(s2 == m2, lane, E), axis=1, keepdims=True)
    wa_ref[...] = m1
    wb_ref[...] = m2
    cnt = ((lane == e0) | (lane == e1)).astype(_F32)
    C = 512
    rr = lax.broadcasted_iota(jnp.int32, (C, C), 0)
    cc = lax.broadcasted_iota(jnp.int32, (C, C), 1)
    tri = (cc < rr).astype(_F32)
    base = jnp.zeros((1, E), _F32)
    excl_parts = []
    for j in range(T // C):
        Cj = lax.slice(cnt, (j * C, 0), ((j + 1) * C, E))
        excl_parts.append(
            lax.dot_general(tri, Cj, (((1,), (0,)), ((), ())),
                            preferred_element_type=_F32) + base)
        base = base + jnp.sum(Cj, axis=0, keepdims=True)
    excl = jnp.concatenate(excl_parts, axis=0)
    counts = base
    pci = ((counts.astype(jnp.int32) + (BLK - 1)) // BLK) * BLK
    pcf = pci.astype(_F32)
    r8 = lax.broadcasted_iota(jnp.int32, (E, E), 0)
    c8 = lax.broadcasted_iota(jnp.int32, (E, E), 1)
    tri8 = (r8 < c8).astype(_F32)
    po = lax.dot_general(pcf, tri8, (((1,), (0,)), ((), ())),
                         preferred_element_type=_F32)
    d0 = jnp.sum(jnp.where(lane == e0, po + excl, 0.0), axis=1, keepdims=True)
    d1 = jnp.sum(jnp.where(lane == e1, po + excl, 0.0), axis=1, keepdims=True)
    d0_ref[...] = d0.astype(jnp.int32)
    d1_ref[...] = d1.astype(jnp.int32)
    ends = (po + pcf).astype(jnp.int32) // BLK
    bb = lax.broadcasted_iota(jnp.int32, (1, NBL), 1)
    eb = jnp.zeros((1, NBL), jnp.int32)
    for e in range(E):
        eb = eb + (bb >= ends[:, e:e + 1]).astype(jnp.int32)
    emap_ref[...] = jnp.minimum(eb, E - 1)
    nb_tot = jnp.sum(pci, axis=1, keepdims=True) // BLK
    act_ref[...] = (bb < nb_tot).astype(jnp.int32)


def _gate(x, Wg):
    return pl.pallas_call(
        _gate_body,
        out_shape=[
            jax.ShapeDtypeStruct((T, 1), _F32),
            jax.ShapeDtypeStruct((T, 1), _F32),
            jax.ShapeDtypeStruct((T, 1), jnp.int32),
            jax.ShapeDtypeStruct((T, 1), jnp.int32),
            jax.ShapeDtypeStruct((1, NBL), jnp.int32),
            jax.ShapeDtypeStruct((1, NBL), jnp.int32),
        ],
    )(x, Wg)


def _sc_scatter(x, d0, d1):
    mesh = plsc.VectorSubcoreMesh(core_axis_name="c", subcore_axis_name="s")
    nw = 32
    tpw = T // nw
    CH = 32

    @functools.partial(
        pl.kernel,
        out_type=jax.ShapeDtypeStruct((NRPAD, DIM), _F32),
        mesh=mesh,
        scratch_types=[
            pltpu.VMEM((CH,), jnp.int32),
            pltpu.VMEM((CH,), jnp.int32),
            pltpu.VMEM((CH, DIM), _F32),
            pltpu.SemaphoreType.DMA,
        ],
    )
    def k(x_hbm, d0_hbm, d1_hbm, xs_hbm, i0_v, i1_v, rows_v, sem):
        wid = lax.axis_index("s") * 2 + lax.axis_index("c")
        for c2 in range(tpw // CH):
            base = wid * tpw + c2 * CH
            pltpu.sync_copy(d0_hbm.at[pl.ds(base, CH)], i0_v)
            pltpu.sync_copy(d1_hbm.at[pl.ds(base, CH)], i1_v)
            pltpu.sync_copy(x_hbm.at[pl.ds(base, CH)], rows_v)
            cp0 = pltpu.async_copy(rows_v, xs_hbm.at[i0_v], sem)
            cp1 = pltpu.async_copy(rows_v, xs_hbm.at[i1_v], sem)
            cp0.wait()
            cp1.wait()

    return k(x, d0, d1)


def _sc_gather(r, d0, d1):
    mesh = plsc.VectorSubcoreMesh(core_axis_name="c", subcore_axis_name="s")
    nw = 32
    tpw = T // nw
    CH = 32

    @functools.partial(
        pl.kernel,
        out_type=[
            jax.ShapeDtypeStruct((T, DIM), _F32),
            jax.ShapeDtypeStruct((T, DIM), _F32),
        ],
        mesh=mesh,
        scratch_types=[
            pltpu.VMEM((CH,), jnp.int32),
            pltpu.VMEM((CH,), jnp.int32),
            pltpu.VMEM((CH, DIM), _F32),
            pltpu.SemaphoreType.DMA,
        ],
    )
    def k(r_hbm, d0_hbm, d1_hbm, o0_hbm, o1_hbm, i0_v, i1_v, rows_v, sem):
        wid = lax.axis_index("s") * 2 + lax.axis_index("c")
        for c2 in range(tpw // CH):
            base = wid * tpw + c2 * CH
            pltpu.sync_copy(d0_hbm.at[pl.ds(base, CH)], i0_v)
            pltpu.async_copy(r_hbm.at[i0_v], rows_v, sem).wait()
            pltpu.sync_copy(rows_v, o0_hbm.at[pl.ds(base, CH)])
            pltpu.sync_copy(d1_hbm.at[pl.ds(base, CH)], i1_v)
            pltpu.async_copy(r_hbm.at[i1_v], rows_v, sem).wait()
            pltpu.sync_copy(rows_v, o1_hbm.at[pl.ds(base, CH)])

    return k(r, d0, d1)


def _mlp_in_body(emap_ref, act_ref, xs_ref, w1_ref, b1_ref, w3_ref, b3_ref, g_ref):
    @pl.when(act_ref[pl.program_id(0)] == 1)
    def _():
        xv = xs_ref[...].astype(_BF)
        h1 = lax.dot_general(xv, w1_ref[0].astype(_BF), (((1,), (1,)), ((), ())),
                             preferred_element_type=_F32) + b1_ref[0]
        h3 = lax.dot_general(xv, w3_ref[0].astype(_BF), (((1,), (1,)), ((), ())),
                             preferred_element_type=_F32) + b3_ref[0]
        g_ref[...] = h1 * _sigmoid(h1) * h3


def _mlp_in(emap, act, xs, W1, b1, W3, b3):
    return pl.pallas_call(
        _mlp_in_body,
        grid_spec=pltpu.PrefetchScalarGridSpec(
            num_scalar_prefetch=2,
            grid=(NB,),
            in_specs=[
                pl.BlockSpec((BLK, DIM), lambda b, em, ac: (b, 0)),
                pl.BlockSpec((1, INTER, DIM), lambda b, em, ac: (em[b], 0, 0)),
                pl.BlockSpec((1, 1, INTER), lambda b, em, ac: (em[b], 0, 0)),
                pl.BlockSpec((1, INTER, DIM), lambda b, em, ac: (em[b], 0, 0)),
                pl.BlockSpec((1, 1, INTER), lambda b, em, ac: (em[b], 0, 0)),
            ],
            out_specs=pl.BlockSpec((BLK, INTER), lambda b, em, ac: (b, 0)),
        ),
        out_shape=jax.ShapeDtypeStruct((NRPAD, INTER), _F32),
    )(emap, act, xs, W1, b1, W3, b3)


def _mlp_out_body(emap_ref, act_ref, g_ref, w2_ref, b2_ref, r_ref):
    @pl.when(act_ref[pl.program_id(0)] == 1)
    def _():
        r_ref[...] = lax.dot_general(g_ref[...].astype(_BF), w2_ref[0].astype(_BF),
                                     (((1,), (1,)), ((), ())),
                                     preferred_element_type=_F32) + b2_ref[0]


def _mlp_out(emap, act, g, W2, b2):
    return pl.pallas_call(
        _mlp_out_body,
        grid_spec=pltpu.PrefetchScalarGridSpec(
            num_scalar_prefetch=2,
            grid=(NB,),
            in_specs=[
                pl.BlockSpec((BLK, INTER), lambda b, em, ac: (b, 0)),
                pl.BlockSpec((1, DIM, INTER), lambda b, em, ac: (em[b], 0, 0)),
                pl.BlockSpec((1, 1, DIM), lambda b, em, ac: (em[b], 0, 0)),
            ],
            out_specs=pl.BlockSpec((BLK, DIM), lambda b, em, ac: (b, 0)),
        ),
        out_shape=jax.ShapeDtypeStruct((NRPAD, DIM), _F32),
    )(emap, act, g, W2, b2)


def _sh_in_body(x_ref, ws1_ref, bs1_ref, ws3_ref, bs3_ref, g_ref):
    xv = x_ref[...].astype(_BF)
    h1 = lax.dot_general(xv, ws1_ref[...].astype(_BF), (((1,), (1,)), ((), ())),
                         preferred_element_type=_F32) + bs1_ref[0]
    h3 = lax.dot_general(xv, ws3_ref[...].astype(_BF), (((1,), (1,)), ((), ())),
                         preferred_element_type=_F32) + bs3_ref[0]
    g_ref[...] = h1 * _sigmoid(h1) * h3


def _sh_in(x, Ws1, bs1r, Ws3, bs3r):
    return pl.pallas_call(
        _sh_in_body,
        grid=(NSH, T // BLK),
        in_specs=[
            pl.BlockSpec((BLK, DIM), lambda i, b: (b, 0)),
            pl.BlockSpec((INTER, DIM), lambda i, b: (i, 0)),
            pl.BlockSpec((1, 1, INTER), lambda i, b: (i, 0, 0)),
            pl.BlockSpec((INTER, DIM), lambda i, b: (i, 0)),
            pl.BlockSpec((1, 1, INTER), lambda i, b: (i, 0, 0)),
        ],
        out_specs=pl.BlockSpec((BLK, INTER), lambda i, b: (b, i)),
        out_shape=jax.ShapeDtypeStruct((T, NSH * INTER), _F32),
    )(x, Ws1, bs1r, Ws3, bs3r)


def _sh_out_body(g_ref, ws2_ref, z_ref):
    z_ref[0] = lax.dot_general(g_ref[...].astype(_BF), ws2_ref[...].astype(_BF),
                               (((1,), (1,)), ((), ())),
                               preferred_element_type=_F32)


def _sh_out(gs, Ws2):
    return pl.pallas_call(
        _sh_out_body,
        grid=(NSH, T // BLK),
        in_specs=[
            pl.BlockSpec((BLK, INTER), lambda i, b: (b, i)),
            pl.BlockSpec((DIM, INTER), lambda i, b: (0, i)),
        ],
        out_specs=pl.BlockSpec((1, BLK, DIM), lambda i, b: (i, b, 0)),
        out_shape=jax.ShapeDtypeStruct((NSH, T, DIM), _F32),
    )(gs, Ws2)


def _combine_body(r0_ref, r1_ref, z_ref, wa_ref, wb_ref, bs2_ref, y_ref):
    y_ref[...] = (wa_ref[...] * r0_ref[...] + wb_ref[...] * r1_ref[...]
                  + z_ref[0] + z_ref[1] + bs2_ref[...])


def _combine(r0, r1, z2, wa, wb, bs2r):
    return pl.pallas_call(
        _combine_body,
        grid=(T // BLK,),
        in_specs=[
            pl.BlockSpec((BLK, DIM), lambda b: (b, 0)),
            pl.BlockSpec((BLK, DIM), lambda b: (b, 0)),
            pl.BlockSpec((NSH, BLK, DIM), lambda b: (0, b, 0)),
            pl.BlockSpec((BLK, 1), lambda b: (b, 0)),
            pl.BlockSpec((BLK, 1), lambda b: (b, 0)),
            pl.BlockSpec((1, DIM), lambda b: (0, 0)),
        ],
        out_specs=pl.BlockSpec((BLK, DIM), lambda b: (b, 0)),
        out_shape=jax.ShapeDtypeStruct((T, DIM), _F32),
    )(r0, r1, z2, wa, wb, bs2r)


def kernel(x, Wg, W1, b1, W2, b2, W3, b3, Ws1, bs1, Ws2, bs2, Ws3, bs3):
    shape = x.shape
    xf = x.reshape(T, DIM)
    wa, wb, d0c, d1c, emap2, act2 = _gate(xf, Wg)
    emap = emap2.reshape(NBL)
    act = act2.reshape(NBL)
    d0 = d0c.reshape(T)
    d1 = d1c.reshape(T)
    xs = _sc_scatter(xf, d0, d1)
    g = _mlp_in(emap, act, xs, W1, b1.reshape(E, 1, INTER), W3, b3.reshape(E, 1, INTER))
    r = _mlp_out(emap, act, g, W2, b2.reshape(E, 1, DIM))
    r0, r1 = _sc_gather(r, d0, d1)
    gs = _sh_in(xf, Ws1, bs1.reshape(NSH, 1, INTER), Ws3, bs3.reshape(NSH, 1, INTER))
    z2 = _sh_out(gs, Ws2)
    y = _combine(r0, r1, z2, wa, wb, bs2.reshape(1, DIM))
    return y.reshape(shape)


# speedup vs baseline: 1.6852x; 1.0318x over previous
import functools

import jax
import jax.numpy as jnp
from jax import lax
from jax.experimental import pallas as pl
from jax.experimental.pallas import tpu as pltpu
from jax.experimental.pallas import tpu_sc as plsc

T = 2048
DIM = 2048
E = 8
INTER = 1408
NSH = 2
BLK = 256
NB = (T * 2) // BLK + E - 1
NBL = 32
NRPAD = NB * BLK
_F32 = jnp.float32
_BF = jnp.bfloat16

_NW = 32
_CH = 16
_TPW = T // _NW
_NC = _TPW // _CH


def _sigmoid(v):
    return 1.0 / (1.0 + jnp.exp(-v))


def _gate_body(x_ref, wg_ref, wa_ref, wb_ref, d0_ref, d1_ref, emap_ref, act_ref):
    xv = x_ref[...]
    logits = lax.dot_general(xv, wg_ref[...], (((1,), (1,)), ((), ())),
                             preferred_element_type=_F32)
    m = jnp.max(logits, axis=1, keepdims=True)
    p = jnp.exp(logits - m)
    scores = p / jnp.sum(p, axis=1, keepdims=True)
    lane = lax.broadcasted_iota(jnp.int32, (T, E), 1)
    m1 = jnp.max(scores, axis=1, keepdims=True)
    e0 = jnp.min(jnp.where(scores == m1, lane, E), axis=1, keepdims=True)
    s2 = jnp.where(lane == e0, -1.0, scores)
    m2 = jnp.max(s2, axis=1, keepdims=True)
    e1 = jnp.min(jnp.where(s2 == m2, lane, E), axis=1, keepdims=True)
    wa_ref[...] = m1
    wb_ref[...] = m2
    cnt = ((lane == e0) | (lane == e1)).astype(_F32)
    C = 512
    rr = lax.broadcasted_iota(jnp.int32, (C, C), 0)
    cc = lax.broadcasted_iota(jnp.int32, (C, C), 1)
    tri = (cc < rr).astype(_F32)
    base = jnp.zeros((1, E), _F32)
    excl_parts = []
    for j in range(T // C):
        Cj = lax.slice(cnt, (j * C, 0), ((j + 1) * C, E))
        excl_parts.append(
            lax.dot_general(tri, Cj, (((1,), (0,)), ((), ())),
                            preferred_element_type=_F32) + base)
        base = base + jnp.sum(Cj, axis=0, keepdims=True)
    excl = jnp.concatenate(excl_parts, axis=0)
    counts = base
    pci = ((counts.astype(jnp.int32) + (BLK - 1)) // BLK) * BLK
    pcf = pci.astype(_F32)
    r8 = lax.broadcasted_iota(jnp.int32, (E, E), 0)
    c8 = lax.broadcasted_iota(jnp.int32, (E, E), 1)
    tri8 = (r8 < c8).astype(_F32)
    po = lax.dot_general(pcf, tri8, (((1,), (0,)), ((), ())),
                         preferred_element_type=_F32)
    d0 = jnp.sum(jnp.where(lane == e0, po + excl, 0.0), axis=1, keepdims=True)
    d1 = jnp.sum(jnp.where(lane == e1, po + excl, 0.0), axis=1, keepdims=True)
    d0_ref[...] = d0.astype(jnp.int32)
    d1_ref[...] = d1.astype(jnp.int32)
    ends = (po + pcf).astype(jnp.int32) // BLK
    bb = lax.broadcasted_iota(jnp.int32, (1, NBL), 1)
    eb = jnp.zeros((1, NBL), jnp.int32)
    for e in range(E):
        eb = eb + (bb >= ends[:, e:e + 1]).astype(jnp.int32)
    emap_ref[...] = jnp.minimum(eb, E - 1)
    nb_tot = jnp.sum(pci, axis=1, keepdims=True) // BLK
    act_ref[...] = (bb < nb_tot).astype(jnp.int32)


def _gate(x, Wg):
    return pl.pallas_call(
        _gate_body,
        out_shape=[
            jax.ShapeDtypeStruct((T, 1), _F32),
            jax.ShapeDtypeStruct((T, 1), _F32),
            jax.ShapeDtypeStruct((T, 1), jnp.int32),
            jax.ShapeDtypeStruct((T, 1), jnp.int32),
            jax.ShapeDtypeStruct((1, NBL), jnp.int32),
            jax.ShapeDtypeStruct((1, NBL), jnp.int32),
        ],
    )(x, Wg)


def _sc_scatter(x, d0r, d1r):
    mesh = plsc.VectorSubcoreMesh(core_axis_name="c", subcore_axis_name="s")

    @functools.partial(
        pl.kernel,
        out_type=jax.ShapeDtypeStruct((NRPAD, DIM), _F32),
        mesh=mesh,
        scratch_types=[
            pltpu.VMEM((2 * _NC, _CH), jnp.int32),
            pltpu.VMEM((2, _CH, DIM), _F32),
            pltpu.SemaphoreType.DMA,
            pltpu.SemaphoreType.DMA,
            pltpu.SemaphoreType.DMA,
            pltpu.SemaphoreType.DMA,
            pltpu.SemaphoreType.DMA,
            pltpu.SemaphoreType.DMA,
        ],
    )
    def k(x_hbm, d0_hbm, d1_hbm, xs_hbm, idx_v, rows_v, sl0, sl1, sa0, sa1, sb0, sb1):
        wid = lax.axis_index("s") * 2 + lax.axis_index("c")
        rbase = wid * _NC
        tbase = wid * _TPW
        pltpu.sync_copy(d0_hbm.at[pl.ds(rbase, _NC)], idx_v.at[pl.ds(0, _NC)])
        pltpu.sync_copy(d1_hbm.at[pl.ds(rbase, _NC)], idx_v.at[pl.ds(_NC, _NC)])
        seml = (sl0, sl1)
        sema = (sa0, sa1)
        semb = (sb0, sb1)
        ld = [None] * _NC
        s0 = [None] * _NC
        s1 = [None] * _NC
        for c in range(2):
            ld[c] = pltpu.async_copy(
                x_hbm.at[pl.ds(tbase + c * _CH, _CH)], rows_v.at[c], seml[c])
        for c in range(_NC):
            buf = c % 2
            ld[c].wait()
            s0[c] = pltpu.async_copy(rows_v.at[buf], xs_hbm.at[idx_v.at[c]], sema[buf])
            s1[c] = pltpu.async_copy(rows_v.at[buf], xs_hbm.at[idx_v.at[_NC + c]], semb[buf])
            if c + 2 < _NC:
                s0[c].wait()
                s1[c].wait()
                ld[c + 2] = pltpu.async_copy(
                    x_hbm.at[pl.ds(tbase + (c + 2) * _CH, _CH)], rows_v.at[buf], seml[buf])
        for c in range(_NC - 2, _NC):
            s0[c].wait()
            s1[c].wait()

    return k(x, d0r, d1r)


def _sc_gather(r, d0r, d1r):
    mesh = plsc.VectorSubcoreMesh(core_axis_name="c", subcore_axis_name="s")

    @functools.partial(
        pl.kernel,
        out_type=[
            jax.ShapeDtypeStruct((T, DIM), _F32),
            jax.ShapeDtypeStruct((T, DIM), _F32),
        ],
        mesh=mesh,
        scratch_types=[
            pltpu.VMEM((2 * _NC, _CH), jnp.int32),
            pltpu.VMEM((2, _CH, DIM), _F32),
            pltpu.SemaphoreType.DMA,
            pltpu.SemaphoreType.DMA,
            pltpu.SemaphoreType.DMA,
            pltpu.SemaphoreType.DMA,
        ],
    )
    def k(r_hbm, d0_hbm, d1_hbm, o0_hbm, o1_hbm, idx_v, rows_v, sg0, sg1, ss0, ss1):
        wid = lax.axis_index("s") * 2 + lax.axis_index("c")
        rbase = wid * _NC
        tbase = wid * _TPW
        pltpu.sync_copy(d0_hbm.at[pl.ds(rbase, _NC)], idx_v.at[pl.ds(0, _NC)])
        pltpu.sync_copy(d1_hbm.at[pl.ds(rbase, _NC)], idx_v.at[pl.ds(_NC, _NC)])
        semg = (sg0, sg1)
        sems = (ss0, ss1)
        outs = (o0_hbm, o1_hbm)
        ntask = 2 * _NC
        g = [None] * ntask
        st = [None] * ntask

        def task_dst(j):
            kslot, c = divmod(j, _NC)
            return outs[kslot].at[pl.ds(tbase + c * _CH, _CH)]

        for j in range(ntask):
            buf = j % 2
            if j >= 2:
                st[j - 2].wait()
            g[j] = pltpu.async_copy(r_hbm.at[idx_v.at[j]], rows_v.at[buf], semg[buf])
            if j >= 1:
                g[j - 1].wait()
                st[j - 1] = pltpu.async_copy(rows_v.at[1 - buf], task_dst(j - 1),
                                             sems[1 - buf])
        g[ntask - 1].wait()
        st[ntask - 1] = pltpu.async_copy(rows_v.at[(ntask - 1) % 2], task_dst(ntask - 1),
                                         sems[(ntask - 1) % 2])
        st[ntask - 2].wait()
        st[ntask - 1].wait()

    return k(r, d0r, d1r)


def _mlp_in_body(emap_ref, act_ref, xs_ref, w1_ref, b1_ref, w3_ref, b3_ref, g_ref):
    @pl.when(act_ref[pl.program_id(0)] == 1)
    def _():
        xv = xs_ref[...].astype(_BF)
        h1 = lax.dot_general(xv, w1_ref[0].astype(_BF), (((1,), (1,)), ((), ())),
                             preferred_element_type=_F32) + b1_ref[0]
        h3 = lax.dot_general(xv, w3_ref[0].astype(_BF), (((1,), (1,)), ((), ())),
                             preferred_element_type=_F32) + b3_ref[0]
        g_ref[...] = (h1 * _sigmoid(h1) * h3).astype(_BF)


def _mlp_in(emap, act, xs, W1, b1, W3, b3):
    return pl.pallas_call(
        _mlp_in_body,
        grid_spec=pltpu.PrefetchScalarGridSpec(
            num_scalar_prefetch=2,
            grid=(NB,),
            in_specs=[
                pl.BlockSpec((BLK, DIM), lambda b, em, ac: (b, 0)),
                pl.BlockSpec((1, INTER, DIM), lambda b, em, ac: (em[b], 0, 0)),
                pl.BlockSpec((1, 1, INTER), lambda b, em, ac: (em[b], 0, 0)),
                pl.BlockSpec((1, INTER, DIM), lambda b, em, ac: (em[b], 0, 0)),
                pl.BlockSpec((1, 1, INTER), lambda b, em, ac: (em[b], 0, 0)),
            ],
            out_specs=pl.BlockSpec((BLK, INTER), lambda b, em, ac: (b, 0)),
        ),
        out_shape=jax.ShapeDtypeStruct((NRPAD, INTER), _BF),
    )(emap, act, xs, W1, b1, W3, b3)


def _mlp_out_body(emap_ref, act_ref, g_ref, w2_ref, b2_ref, r_ref):
    @pl.when(act_ref[pl.program_id(0)] == 1)
    def _():
        r_ref[...] = lax.dot_general(g_ref[...], w2_ref[0].astype(_BF),
                                     (((1,), (1,)), ((), ())),
                                     preferred_element_type=_F32) + b2_ref[0]


def _mlp_out(emap, act, g, W2, b2):
    return pl.pallas_call(
        _mlp_out_body,
        grid_spec=pltpu.PrefetchScalarGridSpec(
            num_scalar_prefetch=2,
            grid=(NB,),
            in_specs=[
                pl.BlockSpec((BLK, INTER), lambda b, em, ac: (b, 0)),
                pl.BlockSpec((1, DIM, INTER), lambda b, em, ac: (em[b], 0, 0)),
                pl.BlockSpec((1, 1, DIM), lambda b, em, ac: (em[b], 0, 0)),
            ],
            out_specs=pl.BlockSpec((BLK, DIM), lambda b, em, ac: (b, 0)),
        ),
        out_shape=jax.ShapeDtypeStruct((NRPAD, DIM), _F32),
    )(emap, act, g, W2, b2)


def _sh_in_body(x_ref, ws1_ref, bs1_ref, ws3_ref, bs3_ref, g_ref):
    xv = x_ref[...].astype(_BF)
    h1 = lax.dot_general(xv, ws1_ref[...].astype(_BF), (((1,), (1,)), ((), ())),
                         preferred_element_type=_F32) + bs1_ref[0]
    h3 = lax.dot_general(xv, ws3_ref[...].astype(_BF), (((1,), (1,)), ((), ())),
                         preferred_element_type=_F32) + bs3_ref[0]
    g_ref[...] = (h1 * _sigmoid(h1) * h3).astype(_BF)


def _sh_in(x, Ws1, bs1r, Ws3, bs3r):
    return pl.pallas_call(
        _sh_in_body,
        grid=(NSH, T // BLK),
        in_specs=[
            pl.BlockSpec((BLK, DIM), lambda i, b: (b, 0)),
            pl.BlockSpec((INTER, DIM), lambda i, b: (i, 0)),
            pl.BlockSpec((1, 1, INTER), lambda i, b: (i, 0, 0)),
            pl.BlockSpec((INTER, DIM), lambda i, b: (i, 0)),
            pl.BlockSpec((1, 1, INTER), lambda i, b: (i, 0, 0)),
        ],
        out_specs=pl.BlockSpec((BLK, INTER), lambda i, b: (b, i)),
        out_shape=jax.ShapeDtypeStruct((T, NSH * INTER), _BF),
    )(x, Ws1, bs1r, Ws3, bs3r)


def _sh_out_body(g_ref, ws2_ref, bs2_ref, z_ref):
    z_ref[...] = lax.dot_general(g_ref[...], ws2_ref[...].astype(_BF),
                                 (((1,), (1,)), ((), ())),
                                 preferred_element_type=_F32) + bs2_ref[...]


def _sh_out(gs, Ws2, bs2r):
    return pl.pallas_call(
        _sh_out_body,
        grid=(T // BLK,),
        in_specs=[
            pl.BlockSpec((BLK, NSH * INTER), lambda b: (b, 0)),
            pl.BlockSpec((DIM, NSH * INTER), lambda b: (0, 0)),
            pl.BlockSpec((1, DIM), lambda b: (0, 0)),
        ],
        out_specs=pl.BlockSpec((BLK, DIM), lambda b: (b, 0)),
        out_shape=jax.ShapeDtypeStruct((T, DIM), _F32),
    )(gs, Ws2, bs2r)


def _combine_body(r0_ref, r1_ref, z_ref, wa_ref, wb_ref, y_ref):
    y_ref[...] = (wa_ref[...] * r0_ref[...] + wb_ref[...] * r1_ref[...]
                  + z_ref[...])


def _combine(r0, r1, z, wa, wb):
    return pl.pallas_call(
        _combine_body,
        grid=(T // BLK,),
        in_specs=[
            pl.BlockSpec((BLK, DIM), lambda b: (b, 0)),
            pl.BlockSpec((BLK, DIM), lambda b: (b, 0)),
            pl.BlockSpec((BLK, DIM), lambda b: (b, 0)),
            pl.BlockSpec((BLK, 1), lambda b: (b, 0)),
            pl.BlockSpec((BLK, 1), lambda b: (b, 0)),
        ],
        out_specs=pl.BlockSpec((BLK, DIM), lambda b: (b, 0)),
        out_shape=jax.ShapeDtypeStruct((T, DIM), _F32),
    )(r0, r1, z, wa, wb)


def kernel(x, Wg, W1, b1, W2, b2, W3, b3, Ws1, bs1, Ws2, bs2, Ws3, bs3):
    shape = x.shape
    xf = x.reshape(T, DIM)
    wa, wb, d0c, d1c, emap2, act2 = _gate(xf, Wg)
    emap = emap2.reshape(NBL)
    act = act2.reshape(NBL)
    d0r = d0c.reshape(T // _CH, _CH)
    d1r = d1c.reshape(T // _CH, _CH)
    xs = _sc_scatter(xf, d0r, d1r)
    g = _mlp_in(emap, act, xs, W1, b1.reshape(E, 1, INTER), W3, b3.reshape(E, 1, INTER))
    r = _mlp_out(emap, act, g, W2, b2.reshape(E, 1, DIM))
    r0, r1 = _sc_gather(r, d0r, d1r)
    gs = _sh_in(xf, Ws1, bs1.reshape(NSH, 1, INTER), Ws3, bs3.reshape(NSH, 1, INTER))
    z = _sh_out(gs, Ws2, bs2.reshape(1, DIM))
    y = _combine(r0, r1, z, wa, wb)
    return y.reshape(shape)
